# initial kernel scaffold (unmeasured)
import jax
import jax.numpy as jnp
from jax import lax
from jax.experimental import pallas as pl
from jax.experimental.pallas import tpu as pltpu

N_DEV = 4


def kernel(x, w_mat, scale_x, scale_w):
    m_per, k = x.shape
    _, n = w_mat.shape
    n_per = n // N_DEV

    def body(x_ref, w_ref, sx_ref, sw_ref, out_ref, ybuf, send_sems, recv_sems):
        my = lax.axis_index("i")

        barrier_sem = pltpu.get_barrier_semaphore()
        for s in range(1, N_DEV):
            pl.semaphore_signal(
                barrier_sem, inc=1,
                device_id=((my + s) % N_DEV,),
                device_id_type=pl.DeviceIdType.MESH,
            )
        pl.semaphore_wait(barrier_sem, N_DEV - 1)

        scale = sx_ref[0] * sw_ref[0]
        acc = jnp.dot(x_ref[...], w_ref[...], preferred_element_type=jnp.float32)
        y = jnp.maximum(acc * scale, 0.0)
        ybuf[...] = y

        out_ref[pl.ds(my * m_per, m_per), :] = lax.dynamic_slice(
            y, (0, my * n_per), (m_per, n_per)
        )

        rdmas = []
        for s in range(1, N_DEV):
            tgt = (my + s) % N_DEV
            rdma = pltpu.make_async_remote_copy(
                src_ref=ybuf.at[:, pl.ds(tgt * n_per, n_per)],
                dst_ref=out_ref.at[pl.ds(my * m_per, m_per), :],
                send_sem=send_sems.at[s],
                recv_sem=recv_sems.at[s],
                device_id=(tgt,),
                device_id_type=pl.DeviceIdType.MESH,
            )
            rdma.start()
            rdmas.append(rdma)
        for rdma in rdmas:
            rdma.wait_send()
        for rdma in rdmas:
            rdma.wait_recv()

    return pl.pallas_call(
        body,
        out_shape=jax.ShapeDtypeStruct((N_DEV * m_per, n_per), jnp.float32),
        in_specs=[
            pl.BlockSpec(memory_space=pltpu.VMEM),
            pl.BlockSpec(memory_space=pltpu.VMEM),
            pl.BlockSpec(memory_space=pltpu.SMEM),
            pl.BlockSpec(memory_space=pltpu.SMEM),
        ],
        out_specs=pl.BlockSpec(memory_space=pltpu.VMEM),
        scratch_shapes=[
            pltpu.VMEM((m_per, n), jnp.float32),
            pltpu.SemaphoreType.DMA((N_DEV,)),
            pltpu.SemaphoreType.DMA((N_DEV,)),
        ],
        compiler_params=pltpu.CompilerParams(collective_id=0),
    )(x, w_mat, scale_x, scale_w)


# baseline (device time: 97257 ns/iter reference)
import jax
import jax.numpy as jnp
from jax import lax
from jax.experimental import pallas as pl
from jax.experimental.pallas import tpu as pltpu

N_DEV = 4


def kernel(x, w_mat, scale_x, scale_w):
    x = x.astype(jnp.float8_e5m2)
    w_mat = w_mat.astype(jnp.float8_e5m2)
    m_per, k = x.shape
    _, n = w_mat.shape
    n_per = n // N_DEV

    def body(x_ref, w_ref, sx_ref, sw_ref, out_ref, ybuf, send_sems, recv_sems):
        my = lax.axis_index("i")

        barrier_sem = pltpu.get_barrier_semaphore()
        for s in range(1, N_DEV):
            pl.semaphore_signal(
                barrier_sem, inc=1,
                device_id=((my + s) % N_DEV,),
                device_id_type=pl.DeviceIdType.MESH,
            )
        pl.semaphore_wait(barrier_sem, N_DEV - 1)

        scale = sx_ref[0] * sw_ref[0]
        acc = jnp.dot(x_ref[...], w_ref[...], preferred_element_type=jnp.float32)
        y = jnp.maximum(acc * scale, 0.0)
        ybuf[...] = y

        out_ref[pl.ds(my * m_per, m_per), :] = ybuf[:, pl.ds(my * n_per, n_per)]

        rdmas = []
        for s in range(1, N_DEV):
            tgt = (my + s) % N_DEV
            rdma = pltpu.make_async_remote_copy(
                src_ref=ybuf.at[:, pl.ds(tgt * n_per, n_per)],
                dst_ref=out_ref.at[pl.ds(my * m_per, m_per), :],
                send_sem=send_sems.at[s],
                recv_sem=recv_sems.at[s],
                device_id=(tgt,),
                device_id_type=pl.DeviceIdType.MESH,
            )
            rdma.start()
            rdmas.append(rdma)
        for rdma in rdmas:
            rdma.wait_send()
        for rdma in rdmas:
            rdma.wait_recv()

    return pl.pallas_call(
        body,
        out_shape=jax.ShapeDtypeStruct((N_DEV * m_per, n_per), jnp.float32),
        in_specs=[
            pl.BlockSpec(memory_space=pltpu.VMEM),
            pl.BlockSpec(memory_space=pltpu.VMEM),
            pl.BlockSpec(memory_space=pltpu.SMEM),
            pl.BlockSpec(memory_space=pltpu.SMEM),
        ],
        out_specs=pl.BlockSpec(memory_space=pltpu.VMEM),
        scratch_shapes=[
            pltpu.VMEM((m_per, n), jnp.float32),
            pltpu.SemaphoreType.DMA((N_DEV,)),
            pltpu.SemaphoreType.DMA((N_DEV,)),
        ],
        compiler_params=pltpu.CompilerParams(
            collective_id=0,
            vmem_limit_bytes=120 * 1024 * 1024,
        ),
    )(x, w_mat, scale_x, scale_w)


# device time: 67960 ns/iter; 1.4311x vs baseline; 1.4311x over previous
import jax
import jax.numpy as jnp
from jax import lax
from jax.experimental import pallas as pl
from jax.experimental.pallas import tpu as pltpu

N_DEV = 4


def kernel(x, w_mat, scale_x, scale_w):
    x = x.astype(jnp.float8_e5m2)
    w_mat = w_mat.astype(jnp.float8_e5m2)
    m_per, k = x.shape
    _, n = w_mat.shape
    n_per = n // N_DEV

    def body(x_ref, w_ref, sx_ref, sw_ref, out_ref, ybuf, rbuf,
             send_sems, recv_sems):
        my = lax.axis_index("i")

        barrier_sem = pltpu.get_barrier_semaphore()
        for s in range(1, N_DEV):
            pl.semaphore_signal(
                barrier_sem, inc=1,
                device_id=((my + s) % N_DEV,),
                device_id_type=pl.DeviceIdType.MESH,
            )
        pl.semaphore_wait(barrier_sem, N_DEV - 1)

        scale = sx_ref[0] * sw_ref[0]
        x_v = x_ref[...]

        def make_rdma(s):
            tgt = (my + s) % N_DEV
            return pltpu.make_async_remote_copy(
                src_ref=ybuf.at[:, pl.ds(((my + s) % N_DEV) * n_per, n_per)],
                dst_ref=rbuf.at[s],
                send_sem=send_sems.at[s],
                recv_sem=recv_sems.at[s],
                device_id=(tgt,),
                device_id_type=pl.DeviceIdType.MESH,
            )

        rdmas = {}
        for s in (2, 1, 3):
            tgt = (my + s) % N_DEV
            blk = jnp.dot(
                x_v, w_ref[:, pl.ds(tgt * n_per, n_per)],
                preferred_element_type=jnp.float32,
            )
            ybuf[:, pl.ds(tgt * n_per, n_per)] = jnp.maximum(
                blk * scale, 0.0
            ).astype(jnp.bfloat16)
            rdmas[s] = make_rdma(s)
            rdmas[s].start()

        blk = jnp.dot(
            x_v, w_ref[:, pl.ds(my * n_per, n_per)],
            preferred_element_type=jnp.float32,
        )
        out_ref[pl.ds(my * m_per, m_per), :] = jnp.maximum(blk * scale, 0.0)

        for s in (1, 3, 2):
            rdmas[s].wait_recv()
            src = (my - s) % N_DEV
            out_ref[pl.ds(src * m_per, m_per), :] = rbuf[s].astype(jnp.float32)
        for s in (1, 2, 3):
            rdmas[s].wait_send()

    return pl.pallas_call(
        body,
        out_shape=jax.ShapeDtypeStruct((N_DEV * m_per, n_per), jnp.float32),
        in_specs=[
            pl.BlockSpec(memory_space=pltpu.VMEM),
            pl.BlockSpec(memory_space=pltpu.VMEM),
            pl.BlockSpec(memory_space=pltpu.SMEM),
            pl.BlockSpec(memory_space=pltpu.SMEM),
        ],
        out_specs=pl.BlockSpec(memory_space=pltpu.VMEM),
        scratch_shapes=[
            pltpu.VMEM((m_per, n), jnp.bfloat16),
            pltpu.VMEM((N_DEV, m_per, n_per), jnp.bfloat16),
            pltpu.SemaphoreType.DMA((N_DEV,)),
            pltpu.SemaphoreType.DMA((N_DEV,)),
        ],
        compiler_params=pltpu.CompilerParams(
            collective_id=0,
            vmem_limit_bytes=56 * 1024 * 1024,
        ),
    )(x, w_mat, scale_x, scale_w)
